# baseline (speedup 1.0000x reference)
import jax
import jax.numpy as jnp
from jax import lax
from jax.experimental import pallas as pl
from jax.experimental.pallas import tpu as pltpu
from jax.experimental.pallas import tpu_sc as plsc

_NUM_EMB = 1000000
_DIM = 64
_BATCH = 4096
_SEQ = 200

_NC = 2
_NS = 16
_NW = _NC * _NS
_BPW = _BATCH // _NW

_C0 = 104
_C1 = _SEQ - _C0
_NPAIR = _BPW // 2


def _body(idx_hbm, table_hbm, out_hbm, idx_v, rows_a, rows_b,
          sem_ga, sem_gb, sem_sa, sem_sb):
  wid = lax.axis_index("s") * _NC + lax.axis_index("c")
  b0 = wid * _BPW
  pltpu.sync_copy(idx_hbm.at[pl.ds(b0, _BPW)], idx_v)

  def fire_gathers(r, rows, sem):
    return [
        pltpu.async_copy(table_hbm.at[idx_v.at[r, pl.ds(0, _C0)]],
                         rows.at[pl.ds(0, _C0)], sem),
        pltpu.async_copy(table_hbm.at[idx_v.at[r, pl.ds(_C0, _C1)]],
                         rows.at[pl.ds(_C0, _C1)], sem),
    ]

  def wait_store(rows, sem):
    pltpu.make_async_copy(rows, out_hbm.at[b0], sem).wait()

  @pl.loop(0, _NPAIR)
  def _pair(i):
    @pl.when(i > 0)
    def _():
      wait_store(rows_a, sem_sa)
      wait_store(rows_b, sem_sb)
    ga = fire_gathers(2 * i, rows_a, sem_ga)
    gb = fire_gathers(2 * i + 1, rows_b, sem_gb)
    for cp in ga:
      cp.wait()
    pltpu.async_copy(rows_a, out_hbm.at[b0 + 2 * i], sem_sa)
    for cp in gb:
      cp.wait()
    pltpu.async_copy(rows_b, out_hbm.at[b0 + 2 * i + 1], sem_sb)

  wait_store(rows_a, sem_sa)
  wait_store(rows_b, sem_sb)


def kernel(x, weight):
  mesh = plsc.VectorSubcoreMesh(
      core_axis_name="c", subcore_axis_name="s",
      num_cores=_NC, num_subcores=_NS)
  return pl.kernel(
      _body,
      out_type=jax.ShapeDtypeStruct((_BATCH, _SEQ, _DIM), jnp.float32),
      mesh=mesh,
      scratch_types=[
          pltpu.VMEM((_BPW, _SEQ), jnp.int32),
          pltpu.VMEM((_SEQ, _DIM), jnp.float32),
          pltpu.VMEM((_SEQ, _DIM), jnp.float32),
          pltpu.SemaphoreType.DMA,
          pltpu.SemaphoreType.DMA,
          pltpu.SemaphoreType.DMA,
          pltpu.SemaphoreType.DMA,
      ],
      compiler_params=pltpu.CompilerParams(use_tc_tiling_on_sc=False),
  )(x.astype(jnp.int32), weight)
